# Initial kernel scaffold; baseline (speedup 1.0000x reference)
import jax
import jax.numpy as jnp
from jax.experimental import pallas as pl


def kernel(x, emb_table, W_feats, b_feats, W_sigma, b_sigma):
    raise NotImplementedError("write your pallas kernel here")


# speedup vs baseline: 1.8256x; 1.8256x over previous
import jax
import jax.numpy as jnp
from jax.experimental import pallas as pl
from jax.experimental.pallas import tpu as pltpu

N_RES = 38
BN = 1000


def _body(x_ref, w_ref, b_ref, o_ref):
    xb = x_ref[...]
    cols = jax.lax.broadcasted_iota(jnp.int32, xb.shape, 1)
    in_head = cols < N_RES
    head = jnp.where(in_head, xb, -jnp.inf)
    mx = jnp.max(head, axis=1, keepdims=True)
    idx = jnp.min(jnp.where(head == mx, cols, jnp.int32(10**9)),
                  axis=1, keepdims=True)
    onehot_or_x = jnp.where(in_head, (cols == idx).astype(xb.dtype), xb)
    o_ref[...] = (
        jnp.dot(onehot_or_x, w_ref[...], preferred_element_type=jnp.float32)
        + b_ref[...]
    )


def kernel(x, emb_table, W_feats, b_feats, W_sigma, b_sigma):
    n, d = x.shape
    n_s = emb_table.shape[1]
    w_full = jnp.concatenate([emb_table, W_feats.T, W_sigma.T], axis=0)
    bias = (b_feats + b_sigma)[None, :]
    return pl.pallas_call(
        _body,
        grid=(n // BN,),
        in_specs=[
            pl.BlockSpec((BN, d), lambda i: (i, 0)),
            pl.BlockSpec((d, n_s), lambda i: (0, 0)),
            pl.BlockSpec((1, n_s), lambda i: (0, 0)),
        ],
        out_specs=pl.BlockSpec((BN, n_s), lambda i: (i, 0)),
        out_shape=jax.ShapeDtypeStruct((n, n_s), jnp.float32),
        compiler_params=pltpu.CompilerParams(
            dimension_semantics=("parallel",),
        ),
    )(x, w_full, bias)


# speedup vs baseline: 2.0588x; 1.1277x over previous
import jax
import jax.numpy as jnp
from jax.experimental import pallas as pl
from jax.experimental.pallas import tpu as pltpu

N_RES = 38
BN = 2000


def _body(x_ref, w_ref, b_ref, o_ref):
    xb = x_ref[...]
    cols = jax.lax.broadcasted_iota(jnp.int32, xb.shape, 1)
    in_head = cols < N_RES
    head = jnp.where(in_head, xb, -jnp.inf)
    mx = jnp.max(head, axis=1, keepdims=True)
    idx = jnp.min(jnp.where(head == mx, cols, jnp.int32(10**9)),
                  axis=1, keepdims=True)
    onehot_or_x = jnp.where(in_head, (cols == idx).astype(xb.dtype), xb)
    o_ref[...] = (
        jnp.dot(onehot_or_x, w_ref[...], preferred_element_type=jnp.float32)
        + b_ref[...]
    )


def kernel(x, emb_table, W_feats, b_feats, W_sigma, b_sigma):
    n, d = x.shape
    n_s = emb_table.shape[1]
    w_full = jnp.concatenate([emb_table, W_feats.T, W_sigma.T], axis=0)
    bias = (b_feats + b_sigma)[None, :]
    return pl.pallas_call(
        _body,
        grid=(n // BN,),
        in_specs=[
            pl.BlockSpec((BN, d), lambda i: (i, 0)),
            pl.BlockSpec((d, n_s), lambda i: (0, 0)),
            pl.BlockSpec((1, n_s), lambda i: (0, 0)),
        ],
        out_specs=pl.BlockSpec((BN, n_s), lambda i: (i, 0)),
        out_shape=jax.ShapeDtypeStruct((n, n_s), jnp.float32),
        compiler_params=pltpu.CompilerParams(
            dimension_semantics=("parallel",),
        ),
    )(x, w_full, bias)


# speedup vs baseline: 2.1921x; 1.0648x over previous
import jax
import jax.numpy as jnp
from jax.experimental import pallas as pl
from jax.experimental.pallas import tpu as pltpu

N_RES = 38
BN = 4000


def _body(x_ref, w_ref, b_ref, o_ref):
    xb = x_ref[...]
    cols = jax.lax.broadcasted_iota(jnp.int32, xb.shape, 1)
    in_head = cols < N_RES
    head = jnp.where(in_head, xb, -jnp.inf)
    mx = jnp.max(head, axis=1, keepdims=True)
    idx = jnp.min(jnp.where(head == mx, cols, jnp.int32(10**9)),
                  axis=1, keepdims=True)
    onehot_or_x = jnp.where(in_head, (cols == idx).astype(xb.dtype), xb)
    o_ref[...] = (
        jnp.dot(onehot_or_x, w_ref[...], preferred_element_type=jnp.float32)
        + b_ref[...]
    )


def kernel(x, emb_table, W_feats, b_feats, W_sigma, b_sigma):
    n, d = x.shape
    n_s = emb_table.shape[1]
    w_full = jnp.concatenate([emb_table, W_feats.T, W_sigma.T], axis=0)
    bias = (b_feats + b_sigma)[None, :]
    return pl.pallas_call(
        _body,
        grid=(n // BN,),
        in_specs=[
            pl.BlockSpec((BN, d), lambda i: (i, 0)),
            pl.BlockSpec((d, n_s), lambda i: (0, 0)),
            pl.BlockSpec((1, n_s), lambda i: (0, 0)),
        ],
        out_specs=pl.BlockSpec((BN, n_s), lambda i: (i, 0)),
        out_shape=jax.ShapeDtypeStruct((n, n_s), jnp.float32),
        compiler_params=pltpu.CompilerParams(
            dimension_semantics=("parallel",),
        ),
    )(x, w_full, bias)


# speedup vs baseline: 2.2000x; 1.0036x over previous
import jax
import jax.numpy as jnp
from jax.experimental import pallas as pl
from jax.experimental.pallas import tpu as pltpu

N_RES = 38
BN = 5000


def _body(x_ref, w_ref, b_ref, o_ref):
    xb = x_ref[...]
    cols = jax.lax.broadcasted_iota(jnp.int32, xb.shape, 1)
    in_head = cols < N_RES
    head = jnp.where(in_head, xb, -jnp.inf)
    mx = jnp.max(head, axis=1, keepdims=True)
    idx = jnp.min(jnp.where(head == mx, cols, jnp.int32(10**9)),
                  axis=1, keepdims=True)
    onehot_or_x = jnp.where(in_head, (cols == idx).astype(xb.dtype), xb)
    o_ref[...] = (
        jnp.dot(onehot_or_x, w_ref[...], preferred_element_type=jnp.float32)
        + b_ref[...]
    )


def kernel(x, emb_table, W_feats, b_feats, W_sigma, b_sigma):
    n, d = x.shape
    n_s = emb_table.shape[1]
    w_full = jnp.concatenate([emb_table, W_feats.T, W_sigma.T], axis=0)
    bias = (b_feats + b_sigma)[None, :]
    return pl.pallas_call(
        _body,
        grid=(n // BN,),
        in_specs=[
            pl.BlockSpec((BN, d), lambda i: (i, 0)),
            pl.BlockSpec((d, n_s), lambda i: (0, 0)),
            pl.BlockSpec((1, n_s), lambda i: (0, 0)),
        ],
        out_specs=pl.BlockSpec((BN, n_s), lambda i: (i, 0)),
        out_shape=jax.ShapeDtypeStruct((n, n_s), jnp.float32),
        compiler_params=pltpu.CompilerParams(
            dimension_semantics=("parallel",),
        ),
    )(x, w_full, bias)
